# Initial kernel scaffold; baseline (speedup 1.0000x reference)
import jax
import jax.numpy as jnp
from jax.experimental import pallas as pl


def kernel(x, batch, W_ih, W_hh, b_ih, b_hh):
    raise NotImplementedError("write your pallas kernel here")


# speedup vs baseline: 5.9373x; 5.9373x over previous
import jax
import jax.numpy as jnp
from jax.experimental import pallas as pl
from jax.experimental.pallas import tpu as pltpu

_N = 50000
_D = 256
_B = 256
_STEPS = 3
_R = 2048
_NBLK = (_N + _R - 1) // _R
_NPAD = _NBLK * _R

_F32 = jnp.float32
_HIGH = jax.lax.Precision.HIGHEST


def _col(v):
    ib = (jax.lax.broadcasted_iota(jnp.int32, (_B, _B), 0) ==
          jax.lax.broadcasted_iota(jnp.int32, (_B, _B), 1))
    return jnp.sum(jnp.where(ib, v, 0.0), axis=1, keepdims=True)


def _body(bat_ref, x_ref, wih_ref, whh_ref, bias_ref, out_ref,
          h_s, c_s, q_s, m_s, s_s, r_s):
    st = pl.program_id(0)
    blk = pl.program_id(1)

    @pl.when(jnp.logical_and(st == 0, blk == 0))
    def _init():
        h_s[...] = jnp.zeros_like(h_s)
        c_s[...] = jnp.zeros_like(c_s)
        q_s[...] = jnp.zeros_like(q_s)

    @pl.when(blk == 0)
    def _lstm():
        q = q_s[...]
        h = h_s[...]
        g = (jax.lax.dot_general(q, wih_ref[...], (((1,), (1,)), ((), ())),
                                 preferred_element_type=_F32)
             + jax.lax.dot_general(h, whh_ref[...], (((1,), (1,)), ((), ())),
                                   preferred_element_type=_F32)
             + bias_ref[...])
        gi = jax.nn.sigmoid(g[:, :_D])
        gf = jax.nn.sigmoid(g[:, _D:2 * _D])
        gg = jnp.tanh(g[:, 2 * _D:3 * _D])
        go = jax.nn.sigmoid(g[:, 3 * _D:])
        c_new = gf * c_s[...] + gi * gg
        h_new = go * jnp.tanh(c_new)
        h_s[...] = h_new
        c_s[...] = c_new
        m_s[...] = jnp.full_like(m_s, -1e30)
        s_s[...] = jnp.zeros_like(s_s)
        r_s[...] = jnp.zeros_like(r_s)

    bat = bat_ref[0]
    iota_b = jax.lax.broadcasted_iota(jnp.int32, (_R, _B), 1)
    pmask = bat == iota_b
    pf = pmask.astype(_F32)
    h = h_s[...]
    qg = jax.lax.dot_general(pf, h, (((1,), (0,)), ((), ())),
                             preferred_element_type=_F32, precision=_HIGH)
    xb = x_ref[...]
    e = jnp.sum(xb * qg, axis=1, keepdims=True)
    em = jnp.where(pmask, e, -1e30)
    mblk = jnp.max(em, axis=0, keepdims=True)
    m_old = m_s[...]
    m_new = jnp.maximum(m_old, mblk)
    scale = jnp.exp(m_old - m_new)
    gm = jnp.sum(jnp.where(pmask, m_new, 0.0), axis=1, keepdims=True)
    ex = jnp.exp(e - gm)
    pw = pf * ex
    sblk = jnp.sum(pw, axis=0, keepdims=True)
    m_s[...] = m_new
    s_s[...] = s_s[...] * scale + sblk
    scale_col = _col(scale)
    racc = jax.lax.dot_general(pw, xb, (((0,), (0,)), ((), ())),
                               preferred_element_type=_F32, precision=_HIGH)
    r_s[...] = r_s[...] * scale_col + racc

    @pl.when(blk == _NBLK - 1)
    def _fin():
        s_col = _col(s_s[...])
        r_fin = r_s[...] / (s_col + 1e-16)
        qnew = jnp.concatenate([h_s[...], r_fin], axis=1)
        q_s[...] = qnew
        out_ref[...] = qnew


def kernel(x, batch, W_ih, W_hh, b_ih, b_hh):
    xp = jnp.pad(x, ((0, _NPAD - _N), (0, 0)))
    batp = jnp.pad(batch, (0, _NPAD - _N),
                   constant_values=_B).reshape(_NBLK, _R, 1)
    bias = (b_ih + b_hh).reshape(1, 4 * _D)
    return pl.pallas_call(
        _body,
        grid=(_STEPS, _NBLK),
        in_specs=[
            pl.BlockSpec((1, _R, 1), lambda s, b: (b, 0, 0)),
            pl.BlockSpec((_R, _D), lambda s, b: (b, 0)),
            pl.BlockSpec((4 * _D, 2 * _D), lambda s, b: (0, 0)),
            pl.BlockSpec((4 * _D, _D), lambda s, b: (0, 0)),
            pl.BlockSpec((1, 4 * _D), lambda s, b: (0, 0)),
        ],
        out_specs=pl.BlockSpec((_B, 2 * _D), lambda s, b: (0, 0)),
        out_shape=jax.ShapeDtypeStruct((_B, 2 * _D), _F32),
        scratch_shapes=[
            pltpu.VMEM((_B, _D), _F32),
            pltpu.VMEM((_B, _D), _F32),
            pltpu.VMEM((_B, 2 * _D), _F32),
            pltpu.VMEM((1, _B), _F32),
            pltpu.VMEM((1, _B), _F32),
            pltpu.VMEM((_B, _D), _F32),
        ],
    )(batp, xp, W_ih, W_hh, bias)


# speedup vs baseline: 11.1181x; 1.8726x over previous
import jax
import jax.numpy as jnp
from jax.experimental import pallas as pl
from jax.experimental.pallas import tpu as pltpu

_N = 50000
_D = 256
_B = 256
_STEPS = 3
_R = 2048
_NBLK = (_N + _R - 1) // _R
_NPAD = _NBLK * _R

_F32 = jnp.float32
_HIGH = jax.lax.Precision.HIGHEST


def _col(v):
    ib = (jax.lax.broadcasted_iota(jnp.int32, (_B, _B), 0) ==
          jax.lax.broadcasted_iota(jnp.int32, (_B, _B), 1))
    return jnp.sum(jnp.where(ib, v, 0.0), axis=1, keepdims=True)


def _body(bat_ref, x_ref, wih_ref, whh_ref, bias_ref, out_ref,
          h_s, c_s, q_s, m_s, s_s, r_s):
    st = pl.program_id(0)
    blk = pl.program_id(1)

    @pl.when(jnp.logical_and(st == 0, blk == 0))
    def _init():
        h_s[...] = jnp.zeros_like(h_s)
        c_s[...] = jnp.zeros_like(c_s)
        q_s[...] = jnp.zeros_like(q_s)

    @pl.when(blk == 0)
    def _lstm():
        q = q_s[...]
        h = h_s[...]
        g = (jax.lax.dot_general(q, wih_ref[...], (((1,), (1,)), ((), ())),
                                 preferred_element_type=_F32)
             + jax.lax.dot_general(h, whh_ref[...], (((1,), (1,)), ((), ())),
                                   preferred_element_type=_F32)
             + bias_ref[...])
        gi = jax.nn.sigmoid(g[:, :_D])
        gf = jax.nn.sigmoid(g[:, _D:2 * _D])
        gg = jnp.tanh(g[:, 2 * _D:3 * _D])
        go = jax.nn.sigmoid(g[:, 3 * _D:])
        c_new = gf * c_s[...] + gi * gg
        h_new = go * jnp.tanh(c_new)
        h_s[...] = h_new
        c_s[...] = c_new
        m_s[...] = jnp.full_like(m_s, -1e30)
        s_s[...] = jnp.zeros_like(s_s)
        r_s[...] = jnp.zeros_like(r_s)

    bat = bat_ref[0]
    iota_b = jax.lax.broadcasted_iota(jnp.int32, (_R, _B), 1)
    pmask = bat == iota_b
    pf = pmask.astype(_F32)
    h = h_s[...]
    qg = jax.lax.dot_general(pf, h, (((1,), (0,)), ((), ())),
                             preferred_element_type=_F32, precision=_HIGH)
    xb = x_ref[...]
    e = jnp.sum(xb * qg, axis=1, keepdims=True)
    em = jnp.where(pmask, e, -1e30)
    mblk = jnp.max(em, axis=0, keepdims=True)
    m_old = m_s[...]
    m_new = jnp.maximum(m_old, mblk)
    scale = jnp.exp(m_old - m_new)
    gm = jnp.sum(jnp.where(pmask, m_new, 0.0), axis=1, keepdims=True)
    ex = jnp.exp(e - gm)
    pw = pf * ex
    sblk = jnp.sum(pw, axis=0, keepdims=True)
    m_s[...] = m_new
    s_s[...] = s_s[...] * scale + sblk
    scale_col = _col(scale)
    racc = jax.lax.dot_general(pw, xb, (((0,), (0,)), ((), ())),
                               preferred_element_type=_F32)
    r_s[...] = r_s[...] * scale_col + racc

    @pl.when(blk == _NBLK - 1)
    def _fin():
        s_col = _col(s_s[...])
        r_fin = r_s[...] / (s_col + 1e-16)
        qnew = jnp.concatenate([h_s[...], r_fin], axis=1)
        q_s[...] = qnew
        out_ref[...] = qnew


def kernel(x, batch, W_ih, W_hh, b_ih, b_hh):
    xp = jnp.pad(x, ((0, _NPAD - _N), (0, 0)))
    batp = jnp.pad(batch, (0, _NPAD - _N),
                   constant_values=_B).reshape(_NBLK, _R, 1)
    bias = (b_ih + b_hh).reshape(1, 4 * _D)
    return pl.pallas_call(
        _body,
        grid=(_STEPS, _NBLK),
        in_specs=[
            pl.BlockSpec((1, _R, 1), lambda s, b: (b, 0, 0)),
            pl.BlockSpec((_R, _D), lambda s, b: (b, 0)),
            pl.BlockSpec((4 * _D, 2 * _D), lambda s, b: (0, 0)),
            pl.BlockSpec((4 * _D, _D), lambda s, b: (0, 0)),
            pl.BlockSpec((1, 4 * _D), lambda s, b: (0, 0)),
        ],
        out_specs=pl.BlockSpec((_B, 2 * _D), lambda s, b: (0, 0)),
        out_shape=jax.ShapeDtypeStruct((_B, 2 * _D), _F32),
        scratch_shapes=[
            pltpu.VMEM((_B, _D), _F32),
            pltpu.VMEM((_B, _D), _F32),
            pltpu.VMEM((_B, 2 * _D), _F32),
            pltpu.VMEM((1, _B), _F32),
            pltpu.VMEM((1, _B), _F32),
            pltpu.VMEM((_B, _D), _F32),
        ],
    )(batp, xp, W_ih, W_hh, bias)


# speedup vs baseline: 15.0148x; 1.3505x over previous
import jax
import jax.numpy as jnp
from jax.experimental import pallas as pl
from jax.experimental.pallas import tpu as pltpu

_N = 50000
_D = 256
_B = 256
_STEPS = 3
_R = 2048
_NBLK = (_N + _R - 1) // _R
_NPAD = _NBLK * _R

_F32 = jnp.float32
_HIGH = jax.lax.Precision.HIGHEST


def _col(v):
    ib = (jax.lax.broadcasted_iota(jnp.int32, (_B, _B), 0) ==
          jax.lax.broadcasted_iota(jnp.int32, (_B, _B), 1))
    return jnp.sum(jnp.where(ib, v, 0.0), axis=1, keepdims=True)


def _body(bat_ref, x_ref, wih_ref, whh_ref, bias_ref, out_ref,
          h_s, c_s, q_s, m_s, s_s, r_s):
    st = pl.program_id(0)
    blk = pl.program_id(1)

    @pl.when(jnp.logical_and(st == 0, blk == 0))
    def _init():
        h_s[...] = jnp.zeros_like(h_s)
        c_s[...] = jnp.zeros_like(c_s)
        q_s[...] = jnp.zeros_like(q_s)

    @pl.when(blk == 0)
    def _lstm():
        q = q_s[...]
        h = h_s[...]
        g = (jax.lax.dot_general(q, wih_ref[...], (((1,), (1,)), ((), ())),
                                 preferred_element_type=_F32)
             + jax.lax.dot_general(h, whh_ref[...], (((1,), (1,)), ((), ())),
                                   preferred_element_type=_F32)
             + bias_ref[...])
        gi = jax.nn.sigmoid(g[:, :_D])
        gf = jax.nn.sigmoid(g[:, _D:2 * _D])
        gg = jnp.tanh(g[:, 2 * _D:3 * _D])
        go = jax.nn.sigmoid(g[:, 3 * _D:])
        c_new = gf * c_s[...] + gi * gg
        h_new = go * jnp.tanh(c_new)
        h_s[...] = h_new
        c_s[...] = c_new
        m_s[...] = jnp.full_like(m_s, -1e30)
        s_s[...] = jnp.zeros_like(s_s)
        r_s[...] = jnp.zeros_like(r_s)

    bat = bat_ref[0]
    iota_b = jax.lax.broadcasted_iota(jnp.int32, (_R, _B), 1)
    pmask = bat == iota_b
    pf = pmask.astype(_F32)
    h = h_s[...]
    h_hi = h.astype(jnp.bfloat16).astype(_F32)
    h_lo = h - h_hi
    qg = (jax.lax.dot_general(pf, h_hi, (((1,), (0,)), ((), ())),
                              preferred_element_type=_F32)
          + jax.lax.dot_general(pf, h_lo, (((1,), (0,)), ((), ())),
                                preferred_element_type=_F32))
    xb = x_ref[...]
    e = jnp.sum(xb * qg, axis=1, keepdims=True)
    em = jnp.where(pmask, e, -1e30)
    mblk = jnp.max(em, axis=0, keepdims=True)
    m_old = m_s[...]
    m_new = jnp.maximum(m_old, mblk)
    scale = jnp.exp(m_old - m_new)
    gm = jnp.sum(jnp.where(pmask, m_new, 0.0), axis=1, keepdims=True)
    ex = jnp.exp(e - gm)
    pw = pf * ex
    sblk = jnp.sum(pw, axis=0, keepdims=True)
    m_s[...] = m_new
    s_s[...] = s_s[...] * scale + sblk
    scale_col = _col(scale)
    racc = jax.lax.dot_general(pw, xb, (((0,), (0,)), ((), ())),
                               preferred_element_type=_F32)
    r_s[...] = r_s[...] * scale_col + racc

    @pl.when(blk == _NBLK - 1)
    def _fin():
        s_col = _col(s_s[...])
        r_fin = r_s[...] / (s_col + 1e-16)
        qnew = jnp.concatenate([h_s[...], r_fin], axis=1)
        q_s[...] = qnew
        out_ref[...] = qnew


def kernel(x, batch, W_ih, W_hh, b_ih, b_hh):
    xp = jnp.pad(x, ((0, _NPAD - _N), (0, 0)))
    batp = jnp.pad(batch, (0, _NPAD - _N),
                   constant_values=_B).reshape(_NBLK, _R, 1)
    bias = (b_ih + b_hh).reshape(1, 4 * _D)
    return pl.pallas_call(
        _body,
        grid=(_STEPS, _NBLK),
        in_specs=[
            pl.BlockSpec((1, _R, 1), lambda s, b: (b, 0, 0)),
            pl.BlockSpec((_R, _D), lambda s, b: (b, 0)),
            pl.BlockSpec((4 * _D, 2 * _D), lambda s, b: (0, 0)),
            pl.BlockSpec((4 * _D, _D), lambda s, b: (0, 0)),
            pl.BlockSpec((1, 4 * _D), lambda s, b: (0, 0)),
        ],
        out_specs=pl.BlockSpec((_B, 2 * _D), lambda s, b: (0, 0)),
        out_shape=jax.ShapeDtypeStruct((_B, 2 * _D), _F32),
        scratch_shapes=[
            pltpu.VMEM((_B, _D), _F32),
            pltpu.VMEM((_B, _D), _F32),
            pltpu.VMEM((_B, 2 * _D), _F32),
            pltpu.VMEM((1, _B), _F32),
            pltpu.VMEM((1, _B), _F32),
            pltpu.VMEM((_B, _D), _F32),
        ],
    )(batp, xp, W_ih, W_hh, bias)
